# baseline (speedup 1.0000x reference)
import functools

import jax
import jax.numpy as jnp
from jax import lax
from jax.experimental import pallas as pl
from jax.experimental.pallas import tpu as pltpu
from jax.experimental.pallas import tpu_sc as plsc

_M = 16384
_K = 8192
_D = 32
_BETA = 0.25
_TM = 256
_TK = 2048
_NK = _K // _TK


def _scan_tile(buf_ref, z2, e2_ref, base, k):
    nl = 128
    rb = 64
    ng = 2
    nj = _TK // nl
    gj = nj // ng
    lane = lax.broadcasted_iota(jnp.int32, (rb, nl), 1)
    tmin_parts = []
    tidx_parts = []
    for r in range(_TM // rb):
        z2r = z2[r * rb:(r + 1) * rb, :]
        accs = []
        for g in range(ng):
            j0 = g * gj
            mv = (z2r + e2_ref[:, j0 * nl:(j0 + 1) * nl]) \
                + buf_ref[pl.ds(base + r * rb, rb), j0 * nl:(j0 + 1) * nl]
            mi = lane + j0 * nl
            for j in range(j0 + 1, j0 + gj):
                d = (z2r + e2_ref[:, j * nl:(j + 1) * nl]) \
                    + buf_ref[pl.ds(base + r * rb, rb), j * nl:(j + 1) * nl]
                upd = d < mv
                mi = jnp.where(upd, lane + j * nl, mi)
                mv = jnp.where(upd, d, mv)
            accs.append((mv, mi))
        mv, mi = accs[0]
        for g in range(1, ng):
            ov, oi = accs[g]
            upd = ov < mv
            mi = jnp.where(upd, oi, mi)
            mv = jnp.where(upd, ov, mv)
        tm = jnp.min(mv, axis=1, keepdims=True)
        ti = jnp.min(jnp.where(mv == tm, mi, _K), axis=1, keepdims=True)
        tmin_parts.append(tm)
        tidx_parts.append(ti)
    tmin = jnp.concatenate(tmin_parts, axis=0)
    tidx = jnp.concatenate(tidx_parts, axis=0) + k * _TK
    return tmin, tidx


def _vq_tc_body(zm2_ref, cb_ref, z2_ref, e2_ref, ema_ref, idx_ref, vq_ref,
                ent_ref, buf_ref, cmin_ref, cidx_ref, gmin_ref, gidx_ref,
                exact_ref, acc_ref):
    s = pl.program_id(0)
    ns = pl.num_programs(0)

    @pl.when(s < ns - 1)
    def _():
        zem = lax.dot_general(zm2_ref[...], cb_ref[...],
                              (((1,), (1,)), ((), ())),
                              preferred_element_type=jnp.float32)
        buf_ref[pl.ds((s % 2) * _TM, _TM), :] = zem

    @pl.when(s == 0)
    def _():
        acc_ref[0] = 0.0
        p = ema_ref[...] + 1e-10
        q = p / jnp.sum(p)
        ent_ref[0] = -jnp.sum(q * jnp.log(q))

    @pl.when(s > 0)
    def _():
        t = s - 1
        k = t % _NK
        tmin, tidx = _scan_tile(buf_ref, z2_ref[...], e2_ref,
                                (t % 2) * _TM, k)

        @pl.when(k % 2 == 0)
        def _():
            cmin_ref[...] = tmin
            cidx_ref[...] = tidx

        @pl.when(k % 2 == 1)
        def _():
            upd = tmin < cmin_ref[...]
            cidx_ref[...] = jnp.where(upd, tidx, cidx_ref[...])
            cmin_ref[...] = jnp.where(upd, tmin, cmin_ref[...])

        @pl.when(k == 0)
        def _():
            exact_ref[...] = tmin

        @pl.when(k > 0)
        def _():
            exact_ref[...] = jnp.minimum(exact_ref[...], tmin)

        @pl.when(k == 1)
        def _():
            gmin_ref[...] = cmin_ref[...].astype(jnp.bfloat16) \
                                         .astype(jnp.float32)
            gidx_ref[...] = cidx_ref[...]

        @pl.when(k == _NK - 1)
        def _():
            upd = cmin_ref[...] < gmin_ref[...]
            idx_ref[...] = jnp.where(upd, cidx_ref[...], gidx_ref[...])
            acc_ref[0] = acc_ref[0] + jnp.sum(exact_ref[...])

            @pl.when(s == ns - 1)
            def _():
                vq_ref[0] = _BETA * acc_ref[0] / (_M * _D)


def _vq_search(z_e, codebook, z2, e2, ema_cluster_size):
    nmk = (_M // _TM) * _NK
    return pl.pallas_call(
        _vq_tc_body,
        grid=(nmk + 1,),
        in_specs=[
            pl.BlockSpec((_TM, _D),
                         lambda s: (jnp.minimum(s, nmk - 1) // _NK, 0)),
            pl.BlockSpec((_TK, _D),
                         lambda s: (jnp.minimum(s, nmk - 1) % _NK, 0)),
            pl.BlockSpec((_TM, 1),
                         lambda s: (jnp.maximum(s - 1, 0) // _NK, 0)),
            pl.BlockSpec((1, _TK),
                         lambda s: (0, jnp.maximum(s - 1, 0) % _NK)),
            pl.BlockSpec((_K,), lambda s: (0,)),
        ],
        out_specs=[
            pl.BlockSpec((_TM, 1),
                         lambda s: (jnp.maximum(s - 1, 0) // _NK, 0)),
            pl.BlockSpec(memory_space=pltpu.SMEM),
            pl.BlockSpec(memory_space=pltpu.SMEM),
        ],
        out_shape=[
            jax.ShapeDtypeStruct((_M, 1), jnp.int32),
            jax.ShapeDtypeStruct((1,), jnp.float32),
            jax.ShapeDtypeStruct((1,), jnp.float32),
        ],
        scratch_shapes=[
            pltpu.VMEM((2 * _TM, _TK), jnp.float32),
            pltpu.VMEM((_TM, 1), jnp.float32),
            pltpu.VMEM((_TM, 1), jnp.int32),
            pltpu.VMEM((_TM, 1), jnp.float32),
            pltpu.VMEM((_TM, 1), jnp.int32),
            pltpu.VMEM((_TM, 1), jnp.float32),
            pltpu.SMEM((1,), jnp.float32),
        ],
        compiler_params=pltpu.CompilerParams(
            dimension_semantics=("arbitrary",)),
    )((-2.0 * z_e).astype(jnp.bfloat16), codebook.astype(jnp.bfloat16),
      z2, e2, ema_cluster_size)


_DP = 128


def _sc_gather(table_pad, indices):
    info = plsc.get_sparse_core_info()
    nc, ns = info.num_cores, info.num_subcores
    nw = nc * ns
    b_per_w = _M // nw
    mesh = plsc.VectorSubcoreMesh(core_axis_name="c", subcore_axis_name="s")

    @functools.partial(
        pl.kernel, mesh=mesh,
        out_type=jax.ShapeDtypeStruct((_M, _DP), jnp.float32),
        scratch_types=[
            pltpu.VMEM((b_per_w,), jnp.int32),
            pltpu.VMEM((b_per_w, _DP), jnp.float32),
            pltpu.SemaphoreType.DMA,
        ],
    )
    def gather_k(table_hbm, idx_hbm, out_hbm, idx_v, rows_v, sem):
        wid = lax.axis_index("s") * nc + lax.axis_index("c")
        base = wid * b_per_w
        pltpu.sync_copy(idx_hbm.at[pl.ds(base, b_per_w)], idx_v)
        pltpu.async_copy(table_hbm.at[idx_v], rows_v, sem).wait()
        pltpu.sync_copy(rows_v, out_hbm.at[pl.ds(base, b_per_w)])

    return gather_k(table_pad, indices)


def kernel(z_e, codebook, ema_cluster_size):
    z2 = (z_e ** 2).sum(axis=1, keepdims=True)
    e2 = (codebook ** 2).sum(axis=1)[None, :]
    idx2d, vq_v, ent_v = _vq_search(z_e, codebook, z2, e2, ema_cluster_size)
    indices = idx2d[:, 0]
    cb_pad = jnp.pad(codebook, ((0, 0), (0, _DP - _D)))
    z_q = _sc_gather(cb_pad, indices)[:, :_D]
    vq_loss = vq_v[0]
    perplexity = jnp.exp(ent_v[0])
    return (z_q, indices, vq_loss, perplexity, z_q)


# speedup vs baseline: 1.1674x; 1.1674x over previous
import functools

import jax
import jax.numpy as jnp
from jax import lax
from jax.experimental import pallas as pl
from jax.experimental.pallas import tpu as pltpu
from jax.experimental.pallas import tpu_sc as plsc

_M = 16384
_K = 8192
_D = 32
_BETA = 0.25
_TM = 256
_TK = 2048
_NK = _K // _TK


def _scan_tile(buf_ref, z2, e2_ref, k):
    nl = 128
    rb = 64
    ng = 2
    nj = _TK // nl
    gj = nj // ng
    lane = lax.broadcasted_iota(jnp.int32, (rb, nl), 1)
    tmin_parts = []
    tidx_parts = []
    for r in range(_TM // rb):
        z2r = z2[r * rb:(r + 1) * rb, :]
        accs = []
        for g in range(ng):
            j0 = g * gj
            mv = (z2r + e2_ref[:, j0 * nl:(j0 + 1) * nl]) \
                + buf_ref[r * rb:(r + 1) * rb, j0 * nl:(j0 + 1) * nl]
            mi = lane + j0 * nl
            for j in range(j0 + 1, j0 + gj):
                d = (z2r + e2_ref[:, j * nl:(j + 1) * nl]) \
                    + buf_ref[r * rb:(r + 1) * rb, j * nl:(j + 1) * nl]
                upd = d < mv
                mi = jnp.where(upd, lane + j * nl, mi)
                mv = jnp.where(upd, d, mv)
            accs.append((mv, mi))
        mv, mi = accs[0]
        for g in range(1, ng):
            ov, oi = accs[g]
            upd = ov < mv
            mi = jnp.where(upd, oi, mi)
            mv = jnp.where(upd, ov, mv)
        tm = jnp.min(mv, axis=1, keepdims=True)
        ti = jnp.min(jnp.where(mv == tm, mi, _K), axis=1, keepdims=True)
        tmin_parts.append(tm)
        tidx_parts.append(ti)
    tmin = jnp.concatenate(tmin_parts, axis=0)
    tidx = jnp.concatenate(tidx_parts, axis=0) + k * _TK
    return tmin, tidx


def _vq_tc_body(zm2_ref, cb_ref, z2_ref, e2_ref, ema_ref, idx_ref, vq_ref,
                ent_ref, bufa_ref, bufb_ref, cmin_ref, cidx_ref, gmin_ref,
                gidx_ref, exact_ref, acc_ref):
    s = pl.program_id(0)
    ns = pl.num_programs(0)

    @pl.when(s == 0)
    def _():
        acc_ref[0] = 0.0
        p = ema_ref[...] + 1e-10
        q = p / jnp.sum(p)
        ent_ref[0] = -jnp.sum(q * jnp.log(q))

    def step(wbuf_ref, rbuf_ref):
        zem = lax.dot_general(zm2_ref[...], cb_ref[...],
                              (((1,), (1,)), ((), ())),
                              preferred_element_type=jnp.float32)
        wbuf_ref[...] = zem

        t = s - 1
        k = t % _NK
        tmin, tidx = _scan_tile(rbuf_ref, z2_ref[...], e2_ref, k)

        @pl.when(k % 2 == 0)
        def _():
            cmin_ref[...] = tmin
            cidx_ref[...] = tidx

        @pl.when(k % 2 == 1)
        def _():
            upd = tmin < cmin_ref[...]
            cidx_ref[...] = jnp.where(upd, tidx, cidx_ref[...])
            cmin_ref[...] = jnp.where(upd, tmin, cmin_ref[...])

        @pl.when(k == 0)
        def _():
            exact_ref[...] = tmin

        @pl.when(k > 0)
        def _():
            exact_ref[...] = jnp.minimum(exact_ref[...], tmin)

        @pl.when(k == 1)
        def _():
            gmin_ref[...] = cmin_ref[...].astype(jnp.bfloat16) \
                                         .astype(jnp.float32)
            gidx_ref[...] = cidx_ref[...]

        @pl.when(k == _NK - 1)
        def _():
            upd = cmin_ref[...] < gmin_ref[...]
            idx_ref[...] = jnp.where(upd, cidx_ref[...], gidx_ref[...])
            acc_ref[0] = acc_ref[0] + jnp.sum(exact_ref[...])

            @pl.when(s == ns - 1)
            def _():
                vq_ref[0] = _BETA * acc_ref[0] / (_M * _D)

    @pl.when(s % 2 == 0)
    def _():
        step(bufa_ref, bufb_ref)

    @pl.when(s % 2 == 1)
    def _():
        step(bufb_ref, bufa_ref)


def _vq_search(z_e, codebook, z2, e2, ema_cluster_size):
    nmk = (_M // _TM) * _NK
    return pl.pallas_call(
        _vq_tc_body,
        grid=(nmk + 1,),
        in_specs=[
            pl.BlockSpec((_TM, _D),
                         lambda s: (jnp.minimum(s, nmk - 1) // _NK, 0)),
            pl.BlockSpec((_TK, _D),
                         lambda s: (jnp.minimum(s, nmk - 1) % _NK, 0)),
            pl.BlockSpec((_TM, 1),
                         lambda s: (jnp.maximum(s - 1, 0) // _NK, 0)),
            pl.BlockSpec((1, _TK),
                         lambda s: (0, jnp.maximum(s - 1, 0) % _NK)),
            pl.BlockSpec((_K,), lambda s: (0,)),
        ],
        out_specs=[
            pl.BlockSpec((_TM, 1),
                         lambda s: (jnp.maximum(s - 1, 0) // _NK, 0)),
            pl.BlockSpec(memory_space=pltpu.SMEM),
            pl.BlockSpec(memory_space=pltpu.SMEM),
        ],
        out_shape=[
            jax.ShapeDtypeStruct((_M, 1), jnp.int32),
            jax.ShapeDtypeStruct((1,), jnp.float32),
            jax.ShapeDtypeStruct((1,), jnp.float32),
        ],
        scratch_shapes=[
            pltpu.VMEM((_TM, _TK), jnp.float32),
            pltpu.VMEM((_TM, _TK), jnp.float32),
            pltpu.VMEM((_TM, 1), jnp.float32),
            pltpu.VMEM((_TM, 1), jnp.int32),
            pltpu.VMEM((_TM, 1), jnp.float32),
            pltpu.VMEM((_TM, 1), jnp.int32),
            pltpu.VMEM((_TM, 1), jnp.float32),
            pltpu.SMEM((1,), jnp.float32),
        ],
        compiler_params=pltpu.CompilerParams(
            dimension_semantics=("arbitrary",)),
    )((-2.0 * z_e).astype(jnp.bfloat16), codebook.astype(jnp.bfloat16),
      z2, e2, ema_cluster_size)


_DP = 128


def _sc_gather(table_pad, indices):
    info = plsc.get_sparse_core_info()
    nc, ns = info.num_cores, info.num_subcores
    nw = nc * ns
    b_per_w = _M // nw
    mesh = plsc.VectorSubcoreMesh(core_axis_name="c", subcore_axis_name="s")

    @functools.partial(
        pl.kernel, mesh=mesh,
        out_type=jax.ShapeDtypeStruct((_M, _DP), jnp.float32),
        scratch_types=[
            pltpu.VMEM((b_per_w,), jnp.int32),
            pltpu.VMEM((b_per_w, _DP), jnp.float32),
            pltpu.SemaphoreType.DMA,
        ],
    )
    def gather_k(table_hbm, idx_hbm, out_hbm, idx_v, rows_v, sem):
        wid = lax.axis_index("s") * nc + lax.axis_index("c")
        base = wid * b_per_w
        pltpu.sync_copy(idx_hbm.at[pl.ds(base, b_per_w)], idx_v)
        pltpu.async_copy(table_hbm.at[idx_v], rows_v, sem).wait()
        pltpu.sync_copy(rows_v, out_hbm.at[pl.ds(base, b_per_w)])

    return gather_k(table_pad, indices)


def kernel(z_e, codebook, ema_cluster_size):
    z2 = (z_e ** 2).sum(axis=1, keepdims=True)
    e2 = (codebook ** 2).sum(axis=1)[None, :]
    idx2d, vq_v, ent_v = _vq_search(z_e, codebook, z2, e2, ema_cluster_size)
    indices = idx2d[:, 0]
    cb_pad = jnp.pad(codebook, ((0, 0), (0, _DP - _D)))
    z_q = _sc_gather(cb_pad, indices)[:, :_D]
    vq_loss = vq_v[0]
    perplexity = jnp.exp(ent_v[0])
    return (z_q, indices, vq_loss, perplexity, z_q)


# speedup vs baseline: 1.8615x; 1.5945x over previous
import functools

import jax
import jax.numpy as jnp
from jax import lax
from jax.experimental import pallas as pl
from jax.experimental.pallas import tpu as pltpu
from jax.experimental.pallas import tpu_sc as plsc

_M = 16384
_K = 8192
_D = 32
_BETA = 0.25
_TM = 256
_TK = 2048
_NK = _K // _TK


def _scan_tile(zem, z2, e2_ref, k):
    nl = 128
    rb = 64
    ng = 2
    nj = _TK // nl
    gj = nj // ng
    c0 = k * _TK
    lane = lax.broadcasted_iota(jnp.int32, (rb, nl), 1)
    tmin_parts = []
    tidx_parts = []
    for r in range(_TM // rb):
        z2r = z2[r * rb:(r + 1) * rb, :]
        accs = []
        for g in range(ng):
            j0 = g * gj
            mv = (z2r + e2_ref[:, c0 + j0 * nl:c0 + (j0 + 1) * nl]) \
                + zem[r * rb:(r + 1) * rb, j0 * nl:(j0 + 1) * nl]
            mi = lane + (c0 + j0 * nl)
            for j in range(j0 + 1, j0 + gj):
                d = (z2r + e2_ref[:, c0 + j * nl:c0 + (j + 1) * nl]) \
                    + zem[r * rb:(r + 1) * rb, j * nl:(j + 1) * nl]
                upd = d < mv
                mi = jnp.where(upd, lane + (c0 + j * nl), mi)
                mv = jnp.where(upd, d, mv)
            accs.append((mv, mi))
        mv, mi = accs[0]
        for g in range(1, ng):
            ov, oi = accs[g]
            upd = ov < mv
            mi = jnp.where(upd, oi, mi)
            mv = jnp.where(upd, ov, mv)
        tm = jnp.min(mv, axis=1, keepdims=True)
        ti = jnp.min(jnp.where(mv == tm, mi, _K), axis=1, keepdims=True)
        tmin_parts.append(tm)
        tidx_parts.append(ti)
    tmin = jnp.concatenate(tmin_parts, axis=0)
    tidx = jnp.concatenate(tidx_parts, axis=0)
    return tmin, tidx


def _vq_tc_body(zm2_ref, cb_ref, z2_ref, e2_ref, ema_ref, idx_ref, vq_ref,
                ent_ref, acc_ref):
    m = pl.program_id(0)
    nm = pl.num_programs(0)

    @pl.when(m == 0)
    def _():
        acc_ref[0] = 0.0
        p = ema_ref[...] + 1e-10
        q = p / jnp.sum(p)
        ent_ref[0] = -jnp.sum(q * jnp.log(q))

    zm2 = zm2_ref[...]
    z2 = z2_ref[...]

    tmins = []
    tidxs = []
    for k in range(_NK):
        cbk = cb_ref[k * _TK:(k + 1) * _TK, :]
        zem = lax.dot_general(zm2, cbk, (((1,), (1,)), ((), ())),
                              preferred_element_type=jnp.float32)
        tm, ti = _scan_tile(zem, z2, e2_ref, k)
        tmins.append(tm)
        tidxs.append(ti)

    def pairmerge(va, ia, vb, ib):
        upd = vb < va
        return jnp.where(upd, vb, va), jnp.where(upd, ib, ia)

    c0v, c0i = pairmerge(tmins[0], tidxs[0], tmins[1], tidxs[1])
    c1v, c1i = pairmerge(tmins[2], tidxs[2], tmins[3], tidxs[3])
    g0v = c0v.astype(jnp.bfloat16).astype(jnp.float32)
    upd = c1v < g0v
    idx_ref[...] = jnp.where(upd, c1i, c0i)

    exact = jnp.minimum(jnp.minimum(tmins[0], tmins[1]),
                        jnp.minimum(tmins[2], tmins[3]))
    acc_ref[0] = acc_ref[0] + jnp.sum(exact)

    @pl.when(m == nm - 1)
    def _():
        vq_ref[0] = _BETA * acc_ref[0] / (_M * _D)


def _vq_search(z_e, codebook, z2, e2, ema_cluster_size):
    return pl.pallas_call(
        _vq_tc_body,
        grid=(_M // _TM,),
        in_specs=[
            pl.BlockSpec((_TM, _D), lambda m: (m, 0)),
            pl.BlockSpec((_K, _D), lambda m: (0, 0)),
            pl.BlockSpec((_TM, 1), lambda m: (m, 0)),
            pl.BlockSpec((1, _K), lambda m: (0, 0)),
            pl.BlockSpec((_K,), lambda m: (0,)),
        ],
        out_specs=[
            pl.BlockSpec((_TM, 1), lambda m: (m, 0)),
            pl.BlockSpec(memory_space=pltpu.SMEM),
            pl.BlockSpec(memory_space=pltpu.SMEM),
        ],
        out_shape=[
            jax.ShapeDtypeStruct((_M, 1), jnp.int32),
            jax.ShapeDtypeStruct((1,), jnp.float32),
            jax.ShapeDtypeStruct((1,), jnp.float32),
        ],
        scratch_shapes=[
            pltpu.SMEM((1,), jnp.float32),
        ],
        compiler_params=pltpu.CompilerParams(
            dimension_semantics=("arbitrary",)),
    )((-2.0 * z_e).astype(jnp.bfloat16), codebook.astype(jnp.bfloat16),
      z2, e2, ema_cluster_size)


_DP = 128


def _sc_gather(table_pad, indices):
    info = plsc.get_sparse_core_info()
    nc, ns = info.num_cores, info.num_subcores
    nw = nc * ns
    b_per_w = _M // nw
    mesh = plsc.VectorSubcoreMesh(core_axis_name="c", subcore_axis_name="s")

    @functools.partial(
        pl.kernel, mesh=mesh,
        out_type=jax.ShapeDtypeStruct((_M, _DP), jnp.float32),
        scratch_types=[
            pltpu.VMEM((b_per_w,), jnp.int32),
            pltpu.VMEM((b_per_w, _DP), jnp.float32),
            pltpu.SemaphoreType.DMA,
        ],
    )
    def gather_k(table_hbm, idx_hbm, out_hbm, idx_v, rows_v, sem):
        wid = lax.axis_index("s") * nc + lax.axis_index("c")
        base = wid * b_per_w
        pltpu.sync_copy(idx_hbm.at[pl.ds(base, b_per_w)], idx_v)
        pltpu.async_copy(table_hbm.at[idx_v], rows_v, sem).wait()
        pltpu.sync_copy(rows_v, out_hbm.at[pl.ds(base, b_per_w)])

    return gather_k(table_pad, indices)


def kernel(z_e, codebook, ema_cluster_size):
    z2 = (z_e ** 2).sum(axis=1, keepdims=True)
    e2 = (codebook ** 2).sum(axis=1)[None, :]
    idx2d, vq_v, ent_v = _vq_search(z_e, codebook, z2, e2, ema_cluster_size)
    indices = idx2d[:, 0]
    cb_pad = jnp.pad(codebook, ((0, 0), (0, _DP - _D)))
    z_q = _sc_gather(cb_pad, indices)[:, :_D]
    vq_loss = vq_v[0]
    perplexity = jnp.exp(ent_v[0])
    return (z_q, indices, vq_loss, perplexity, z_q)


# speedup vs baseline: 1.8672x; 1.0031x over previous
import functools

import jax
import jax.numpy as jnp
from jax import lax
from jax.experimental import pallas as pl
from jax.experimental.pallas import tpu as pltpu
from jax.experimental.pallas import tpu_sc as plsc

_M = 16384
_K = 8192
_D = 32
_BETA = 0.25
_TM = 256
_TK = 2048
_NK = _K // _TK


def _scan_tile(zem, z2, e2_ref, k):
    nl = 128
    rb = 64
    ng = 4
    nj = _TK // nl
    gj = nj // ng
    c0 = k * _TK
    lane = lax.broadcasted_iota(jnp.int32, (rb, nl), 1)
    tmin_parts = []
    tidx_parts = []
    for r in range(_TM // rb):
        z2r = z2[r * rb:(r + 1) * rb, :]
        accs = []
        for g in range(ng):
            j0 = g * gj
            mv = (z2r + e2_ref[:, c0 + j0 * nl:c0 + (j0 + 1) * nl]) \
                + zem[r * rb:(r + 1) * rb, j0 * nl:(j0 + 1) * nl]
            mi = lane + (c0 + j0 * nl)
            for j in range(j0 + 1, j0 + gj):
                d = (z2r + e2_ref[:, c0 + j * nl:c0 + (j + 1) * nl]) \
                    + zem[r * rb:(r + 1) * rb, j * nl:(j + 1) * nl]
                upd = d < mv
                mi = jnp.where(upd, lane + (c0 + j * nl), mi)
                mv = jnp.where(upd, d, mv)
            accs.append((mv, mi))
        mv, mi = accs[0]
        for g in range(1, ng):
            ov, oi = accs[g]
            upd = ov < mv
            mi = jnp.where(upd, oi, mi)
            mv = jnp.where(upd, ov, mv)
        tm = jnp.min(mv, axis=1, keepdims=True)
        ti = jnp.min(jnp.where(mv == tm, mi, _K), axis=1, keepdims=True)
        tmin_parts.append(tm)
        tidx_parts.append(ti)
    tmin = jnp.concatenate(tmin_parts, axis=0)
    tidx = jnp.concatenate(tidx_parts, axis=0)
    return tmin, tidx


def _vq_tc_body(zm2_ref, cb_ref, z2_ref, e2_ref, ema_ref, idx_ref, vq_ref,
                ent_ref, acc_ref):
    m = pl.program_id(0)
    nm = pl.num_programs(0)

    @pl.when(m == 0)
    def _():
        acc_ref[0] = 0.0
        p = ema_ref[...] + 1e-10
        q = p / jnp.sum(p)
        ent_ref[0] = -jnp.sum(q * jnp.log(q))

    zm2 = zm2_ref[...]
    z2 = z2_ref[...]

    tmins = []
    tidxs = []
    for k in range(_NK):
        cbk = cb_ref[k * _TK:(k + 1) * _TK, :]
        zem = lax.dot_general(zm2, cbk, (((1,), (1,)), ((), ())),
                              preferred_element_type=jnp.float32)
        tm, ti = _scan_tile(zem, z2, e2_ref, k)
        tmins.append(tm)
        tidxs.append(ti)

    def pairmerge(va, ia, vb, ib):
        upd = vb < va
        return jnp.where(upd, vb, va), jnp.where(upd, ib, ia)

    c0v, c0i = pairmerge(tmins[0], tidxs[0], tmins[1], tidxs[1])
    c1v, c1i = pairmerge(tmins[2], tidxs[2], tmins[3], tidxs[3])
    g0v = c0v.astype(jnp.bfloat16).astype(jnp.float32)
    upd = c1v < g0v
    idx_ref[...] = jnp.where(upd, c1i, c0i)

    exact = jnp.minimum(jnp.minimum(tmins[0], tmins[1]),
                        jnp.minimum(tmins[2], tmins[3]))
    acc_ref[0] = acc_ref[0] + jnp.sum(exact)

    @pl.when(m == nm - 1)
    def _():
        vq_ref[0] = _BETA * acc_ref[0] / (_M * _D)


def _vq_search(z_e, codebook, z2, e2, ema_cluster_size):
    return pl.pallas_call(
        _vq_tc_body,
        grid=(_M // _TM,),
        in_specs=[
            pl.BlockSpec((_TM, _D), lambda m: (m, 0)),
            pl.BlockSpec((_K, _D), lambda m: (0, 0)),
            pl.BlockSpec((_TM, 1), lambda m: (m, 0)),
            pl.BlockSpec((1, _K), lambda m: (0, 0)),
            pl.BlockSpec((_K,), lambda m: (0,)),
        ],
        out_specs=[
            pl.BlockSpec((_TM, 1), lambda m: (m, 0)),
            pl.BlockSpec(memory_space=pltpu.SMEM),
            pl.BlockSpec(memory_space=pltpu.SMEM),
        ],
        out_shape=[
            jax.ShapeDtypeStruct((_M, 1), jnp.int32),
            jax.ShapeDtypeStruct((1,), jnp.float32),
            jax.ShapeDtypeStruct((1,), jnp.float32),
        ],
        scratch_shapes=[
            pltpu.SMEM((1,), jnp.float32),
        ],
        compiler_params=pltpu.CompilerParams(
            dimension_semantics=("arbitrary",)),
    )((-2.0 * z_e).astype(jnp.bfloat16), codebook.astype(jnp.bfloat16),
      z2, e2, ema_cluster_size)


_DP = 128


def _sc_gather(table_pad, indices):
    info = plsc.get_sparse_core_info()
    nc, ns = info.num_cores, info.num_subcores
    nw = nc * ns
    b_per_w = _M // nw
    mesh = plsc.VectorSubcoreMesh(core_axis_name="c", subcore_axis_name="s")

    @functools.partial(
        pl.kernel, mesh=mesh,
        out_type=jax.ShapeDtypeStruct((_M, _DP), jnp.float32),
        scratch_types=[
            pltpu.VMEM((b_per_w,), jnp.int32),
            pltpu.VMEM((b_per_w, _DP), jnp.float32),
            pltpu.SemaphoreType.DMA,
        ],
    )
    def gather_k(table_hbm, idx_hbm, out_hbm, idx_v, rows_v, sem):
        wid = lax.axis_index("s") * nc + lax.axis_index("c")
        base = wid * b_per_w
        pltpu.sync_copy(idx_hbm.at[pl.ds(base, b_per_w)], idx_v)
        pltpu.async_copy(table_hbm.at[idx_v], rows_v, sem).wait()
        pltpu.sync_copy(rows_v, out_hbm.at[pl.ds(base, b_per_w)])

    return gather_k(table_pad, indices)


def kernel(z_e, codebook, ema_cluster_size):
    z2 = (z_e ** 2).sum(axis=1, keepdims=True)
    e2 = (codebook ** 2).sum(axis=1)[None, :]
    idx2d, vq_v, ent_v = _vq_search(z_e, codebook, z2, e2, ema_cluster_size)
    indices = idx2d[:, 0]
    cb_pad = jnp.pad(codebook, ((0, 0), (0, _DP - _D)))
    z_q = _sc_gather(cb_pad, indices)[:, :_D]
    vq_loss = vq_v[0]
    perplexity = jnp.exp(ent_v[0])
    return (z_q, indices, vq_loss, perplexity, z_q)


# speedup vs baseline: 2.1156x; 1.1330x over previous
import functools

import jax
import jax.numpy as jnp
from jax import lax
from jax.experimental import pallas as pl
from jax.experimental.pallas import tpu as pltpu
from jax.experimental.pallas import tpu_sc as plsc

_M = 16384
_K = 8192
_D = 32
_BETA = 0.25
_TM = 512
_TK = 2048
_NK = _K // _TK


def _scan_tile(zem, z2, e2_ref, k):
    nl = 128
    rb = 64
    ng = 4
    nj = _TK // nl
    gj = nj // ng
    c0 = k * _TK
    lane = lax.broadcasted_iota(jnp.int32, (rb, nl), 1)
    tmin_parts = []
    tidx_parts = []
    for r in range(_TM // rb):
        z2r = z2[r * rb:(r + 1) * rb, :]
        accs = []
        for g in range(ng):
            j0 = g * gj
            mv = (z2r + e2_ref[:, c0 + j0 * nl:c0 + (j0 + 1) * nl]) \
                + zem[r * rb:(r + 1) * rb, j0 * nl:(j0 + 1) * nl]
            mi = lane + (c0 + j0 * nl)
            for j in range(j0 + 1, j0 + gj):
                d = (z2r + e2_ref[:, c0 + j * nl:c0 + (j + 1) * nl]) \
                    + zem[r * rb:(r + 1) * rb, j * nl:(j + 1) * nl]
                upd = d < mv
                mi = jnp.where(upd, lane + (c0 + j * nl), mi)
                mv = jnp.where(upd, d, mv)
            accs.append((mv, mi))
        mv, mi = accs[0]
        for g in range(1, ng):
            ov, oi = accs[g]
            upd = ov < mv
            mi = jnp.where(upd, oi, mi)
            mv = jnp.where(upd, ov, mv)
        tm = jnp.min(mv, axis=1, keepdims=True)
        ti = jnp.min(jnp.where(mv == tm, mi, _K), axis=1, keepdims=True)
        tmin_parts.append(tm)
        tidx_parts.append(ti)
    tmin = jnp.concatenate(tmin_parts, axis=0)
    tidx = jnp.concatenate(tidx_parts, axis=0)
    return tmin, tidx


def _vq_tc_body(zm2_ref, cb_ref, z2_ref, e2_ref, ema_ref, idx_ref, vq_ref,
                ent_ref, acc_ref):
    m = pl.program_id(0)
    nm = pl.num_programs(0)

    @pl.when(m == 0)
    def _():
        acc_ref[0] = 0.0
        p = ema_ref[...] + 1e-10
        q = p / jnp.sum(p)
        ent_ref[0] = -jnp.sum(q * jnp.log(q))

    zm2 = zm2_ref[...]
    z2 = z2_ref[...]

    tmins = []
    tidxs = []
    for k in range(_NK):
        cbk = cb_ref[k * _TK:(k + 1) * _TK, :]
        zem = lax.dot_general(zm2, cbk, (((1,), (1,)), ((), ())),
                              preferred_element_type=jnp.float32)
        tm, ti = _scan_tile(zem, z2, e2_ref, k)
        tmins.append(tm)
        tidxs.append(ti)

    def pairmerge(va, ia, vb, ib):
        upd = vb < va
        return jnp.where(upd, vb, va), jnp.where(upd, ib, ia)

    c0v, c0i = pairmerge(tmins[0], tidxs[0], tmins[1], tidxs[1])
    c1v, c1i = pairmerge(tmins[2], tidxs[2], tmins[3], tidxs[3])
    g0v = c0v.astype(jnp.bfloat16).astype(jnp.float32)
    upd = c1v < g0v
    idx_ref[...] = jnp.where(upd, c1i, c0i)

    exact = jnp.minimum(jnp.minimum(tmins[0], tmins[1]),
                        jnp.minimum(tmins[2], tmins[3]))
    acc_ref[0] = acc_ref[0] + jnp.sum(exact)

    @pl.when(m == nm - 1)
    def _():
        vq_ref[0] = _BETA * acc_ref[0] / (_M * _D)


def _vq_search(z_e, codebook, z2, e2, ema_cluster_size):
    return pl.pallas_call(
        _vq_tc_body,
        grid=(_M // _TM,),
        in_specs=[
            pl.BlockSpec((_TM, _D), lambda m: (m, 0)),
            pl.BlockSpec((_K, _D), lambda m: (0, 0)),
            pl.BlockSpec((_TM, 1), lambda m: (m, 0)),
            pl.BlockSpec((1, _K), lambda m: (0, 0)),
            pl.BlockSpec((_K,), lambda m: (0,)),
        ],
        out_specs=[
            pl.BlockSpec((_TM, 1), lambda m: (m, 0)),
            pl.BlockSpec(memory_space=pltpu.SMEM),
            pl.BlockSpec(memory_space=pltpu.SMEM),
        ],
        out_shape=[
            jax.ShapeDtypeStruct((_M, 1), jnp.int32),
            jax.ShapeDtypeStruct((1,), jnp.float32),
            jax.ShapeDtypeStruct((1,), jnp.float32),
        ],
        scratch_shapes=[
            pltpu.SMEM((1,), jnp.float32),
        ],
        compiler_params=pltpu.CompilerParams(
            dimension_semantics=("arbitrary",)),
    )((-2.0 * z_e).astype(jnp.bfloat16), codebook.astype(jnp.bfloat16),
      z2, e2, ema_cluster_size)


_DP = 128


def _sc_gather(table_pad, indices):
    info = plsc.get_sparse_core_info()
    nc, ns = info.num_cores, info.num_subcores
    nw = nc * ns
    b_per_w = _M // nw
    mesh = plsc.VectorSubcoreMesh(core_axis_name="c", subcore_axis_name="s")

    @functools.partial(
        pl.kernel, mesh=mesh,
        out_type=jax.ShapeDtypeStruct((_M, _DP), jnp.float32),
        scratch_types=[
            pltpu.VMEM((b_per_w,), jnp.int32),
            pltpu.VMEM((b_per_w, _DP), jnp.float32),
            pltpu.SemaphoreType.DMA,
        ],
    )
    def gather_k(table_hbm, idx_hbm, out_hbm, idx_v, rows_v, sem):
        wid = lax.axis_index("s") * nc + lax.axis_index("c")
        base = wid * b_per_w
        pltpu.sync_copy(idx_hbm.at[pl.ds(base, b_per_w)], idx_v)
        pltpu.async_copy(table_hbm.at[idx_v], rows_v, sem).wait()
        pltpu.sync_copy(rows_v, out_hbm.at[pl.ds(base, b_per_w)])

    return gather_k(table_pad, indices)


def kernel(z_e, codebook, ema_cluster_size):
    z2 = (z_e ** 2).sum(axis=1, keepdims=True)
    e2 = (codebook ** 2).sum(axis=1)[None, :]
    idx2d, vq_v, ent_v = _vq_search(z_e, codebook, z2, e2, ema_cluster_size)
    indices = idx2d[:, 0]
    cb_pad = jnp.pad(codebook, ((0, 0), (0, _DP - _D)))
    z_q = _sc_gather(cb_pad, indices)[:, :_D]
    vq_loss = vq_v[0]
    perplexity = jnp.exp(ent_v[0])
    return (z_q, indices, vq_loss, perplexity, z_q)


# speedup vs baseline: 2.1845x; 1.0326x over previous
import functools

import jax
import jax.numpy as jnp
from jax import lax
from jax.experimental import pallas as pl
from jax.experimental.pallas import tpu as pltpu
from jax.experimental.pallas import tpu_sc as plsc

_M = 16384
_K = 8192
_D = 32
_BETA = 0.25
_TM = 1024
_TK = 2048
_NK = _K // _TK


def _scan_tile(zem, z2, e2_ref, k):
    nl = 128
    rb = 64
    ng = 4
    nj = _TK // nl
    gj = nj // ng
    c0 = k * _TK
    lane = lax.broadcasted_iota(jnp.int32, (rb, nl), 1)
    tmin_parts = []
    tidx_parts = []
    for r in range(_TM // rb):
        z2r = z2[r * rb:(r + 1) * rb, :]
        accs = []
        for g in range(ng):
            j0 = g * gj
            mv = (z2r + e2_ref[:, c0 + j0 * nl:c0 + (j0 + 1) * nl]) \
                + zem[r * rb:(r + 1) * rb, j0 * nl:(j0 + 1) * nl]
            mi = lane + (c0 + j0 * nl)
            for j in range(j0 + 1, j0 + gj):
                d = (z2r + e2_ref[:, c0 + j * nl:c0 + (j + 1) * nl]) \
                    + zem[r * rb:(r + 1) * rb, j * nl:(j + 1) * nl]
                upd = d < mv
                mi = jnp.where(upd, lane + (c0 + j * nl), mi)
                mv = jnp.where(upd, d, mv)
            accs.append((mv, mi))
        mv, mi = accs[0]
        for g in range(1, ng):
            ov, oi = accs[g]
            upd = ov < mv
            mi = jnp.where(upd, oi, mi)
            mv = jnp.where(upd, ov, mv)
        tm = jnp.min(mv, axis=1, keepdims=True)
        ti = jnp.min(jnp.where(mv == tm, mi, _K), axis=1, keepdims=True)
        tmin_parts.append(tm)
        tidx_parts.append(ti)
    tmin = jnp.concatenate(tmin_parts, axis=0)
    tidx = jnp.concatenate(tidx_parts, axis=0)
    return tmin, tidx


def _vq_tc_body(zm2_ref, cb_ref, z2_ref, e2_ref, ema_ref, idx_ref, vq_ref,
                ent_ref, acc_ref):
    m = pl.program_id(0)
    nm = pl.num_programs(0)

    @pl.when(m == 0)
    def _():
        acc_ref[0] = 0.0
        p = ema_ref[...] + 1e-10
        q = p / jnp.sum(p)
        ent_ref[0] = -jnp.sum(q * jnp.log(q))

    zm2 = zm2_ref[...]
    z2 = z2_ref[...]

    tmins = []
    tidxs = []
    for k in range(_NK):
        cbk = cb_ref[k * _TK:(k + 1) * _TK, :]
        zem = lax.dot_general(zm2, cbk, (((1,), (1,)), ((), ())),
                              preferred_element_type=jnp.float32)
        tm, ti = _scan_tile(zem, z2, e2_ref, k)
        tmins.append(tm)
        tidxs.append(ti)

    def pairmerge(va, ia, vb, ib):
        upd = vb < va
        return jnp.where(upd, vb, va), jnp.where(upd, ib, ia)

    c0v, c0i = pairmerge(tmins[0], tidxs[0], tmins[1], tidxs[1])
    c1v, c1i = pairmerge(tmins[2], tidxs[2], tmins[3], tidxs[3])
    g0v = c0v.astype(jnp.bfloat16).astype(jnp.float32)
    upd = c1v < g0v
    idx_ref[...] = jnp.where(upd, c1i, c0i)

    exact = jnp.minimum(jnp.minimum(tmins[0], tmins[1]),
                        jnp.minimum(tmins[2], tmins[3]))
    acc_ref[0] = acc_ref[0] + jnp.sum(exact)

    @pl.when(m == nm - 1)
    def _():
        vq_ref[0] = _BETA * acc_ref[0] / (_M * _D)


def _vq_search(z_e, codebook, z2, e2, ema_cluster_size):
    return pl.pallas_call(
        _vq_tc_body,
        grid=(_M // _TM,),
        in_specs=[
            pl.BlockSpec((_TM, _D), lambda m: (m, 0)),
            pl.BlockSpec((_K, _D), lambda m: (0, 0)),
            pl.BlockSpec((_TM, 1), lambda m: (m, 0)),
            pl.BlockSpec((1, _K), lambda m: (0, 0)),
            pl.BlockSpec((_K,), lambda m: (0,)),
        ],
        out_specs=[
            pl.BlockSpec((_TM, 1), lambda m: (m, 0)),
            pl.BlockSpec(memory_space=pltpu.SMEM),
            pl.BlockSpec(memory_space=pltpu.SMEM),
        ],
        out_shape=[
            jax.ShapeDtypeStruct((_M, 1), jnp.int32),
            jax.ShapeDtypeStruct((1,), jnp.float32),
            jax.ShapeDtypeStruct((1,), jnp.float32),
        ],
        scratch_shapes=[
            pltpu.SMEM((1,), jnp.float32),
        ],
        compiler_params=pltpu.CompilerParams(
            dimension_semantics=("arbitrary",)),
    )((-2.0 * z_e).astype(jnp.bfloat16), codebook.astype(jnp.bfloat16),
      z2, e2, ema_cluster_size)


_DP = 128


def _sc_gather(table_pad, indices):
    info = plsc.get_sparse_core_info()
    nc, ns = info.num_cores, info.num_subcores
    nw = nc * ns
    b_per_w = _M // nw
    mesh = plsc.VectorSubcoreMesh(core_axis_name="c", subcore_axis_name="s")

    @functools.partial(
        pl.kernel, mesh=mesh,
        out_type=jax.ShapeDtypeStruct((_M, _DP), jnp.float32),
        scratch_types=[
            pltpu.VMEM((b_per_w,), jnp.int32),
            pltpu.VMEM((b_per_w, _DP), jnp.float32),
            pltpu.SemaphoreType.DMA,
        ],
    )
    def gather_k(table_hbm, idx_hbm, out_hbm, idx_v, rows_v, sem):
        wid = lax.axis_index("s") * nc + lax.axis_index("c")
        base = wid * b_per_w
        pltpu.sync_copy(idx_hbm.at[pl.ds(base, b_per_w)], idx_v)
        pltpu.async_copy(table_hbm.at[idx_v], rows_v, sem).wait()
        pltpu.sync_copy(rows_v, out_hbm.at[pl.ds(base, b_per_w)])

    return gather_k(table_pad, indices)


def kernel(z_e, codebook, ema_cluster_size):
    z2 = (z_e ** 2).sum(axis=1, keepdims=True)
    e2 = (codebook ** 2).sum(axis=1)[None, :]
    idx2d, vq_v, ent_v = _vq_search(z_e, codebook, z2, e2, ema_cluster_size)
    indices = idx2d[:, 0]
    cb_pad = jnp.pad(codebook, ((0, 0), (0, _DP - _D)))
    z_q = _sc_gather(cb_pad, indices)[:, :_D]
    vq_loss = vq_v[0]
    perplexity = jnp.exp(ent_v[0])
    return (z_q, indices, vq_loss, perplexity, z_q)


# speedup vs baseline: 2.2399x; 1.0253x over previous
import functools

import jax
import jax.numpy as jnp
from jax import lax
from jax.experimental import pallas as pl
from jax.experimental.pallas import tpu as pltpu
from jax.experimental.pallas import tpu_sc as plsc

_M = 16384
_K = 8192
_D = 32
_BETA = 0.25
_TM = 2048
_TK = 2048
_NK = _K // _TK


def _scan_tile(zem, z2, e2_ref, k):
    nl = 128
    rb = 64
    ng = 4
    nj = _TK // nl
    gj = nj // ng
    c0 = k * _TK
    lane = lax.broadcasted_iota(jnp.int32, (rb, nl), 1)
    tmin_parts = []
    tidx_parts = []
    for r in range(_TM // rb):
        z2r = z2[r * rb:(r + 1) * rb, :]
        accs = []
        for g in range(ng):
            j0 = g * gj
            mv = (z2r + e2_ref[:, c0 + j0 * nl:c0 + (j0 + 1) * nl]) \
                + zem[r * rb:(r + 1) * rb, j0 * nl:(j0 + 1) * nl]
            mi = lane + (c0 + j0 * nl)
            for j in range(j0 + 1, j0 + gj):
                d = (z2r + e2_ref[:, c0 + j * nl:c0 + (j + 1) * nl]) \
                    + zem[r * rb:(r + 1) * rb, j * nl:(j + 1) * nl]
                upd = d < mv
                mi = jnp.where(upd, lane + (c0 + j * nl), mi)
                mv = jnp.where(upd, d, mv)
            accs.append((mv, mi))
        mv, mi = accs[0]
        for g in range(1, ng):
            ov, oi = accs[g]
            upd = ov < mv
            mi = jnp.where(upd, oi, mi)
            mv = jnp.where(upd, ov, mv)
        tm = jnp.min(mv, axis=1, keepdims=True)
        ti = jnp.min(jnp.where(mv == tm, mi, _K), axis=1, keepdims=True)
        tmin_parts.append(tm)
        tidx_parts.append(ti)
    tmin = jnp.concatenate(tmin_parts, axis=0)
    tidx = jnp.concatenate(tidx_parts, axis=0)
    return tmin, tidx


def _vq_tc_body(zm2_ref, cb_ref, z2_ref, e2_ref, ema_ref, idx_ref, vq_ref,
                ent_ref, acc_ref):
    m = pl.program_id(0)
    nm = pl.num_programs(0)

    @pl.when(m == 0)
    def _():
        acc_ref[0] = 0.0
        p = ema_ref[...] + 1e-10
        q = p / jnp.sum(p)
        ent_ref[0] = -jnp.sum(q * jnp.log(q))

    zm2 = zm2_ref[...]
    z2 = z2_ref[...]

    tmins = []
    tidxs = []
    for k in range(_NK):
        cbk = cb_ref[k * _TK:(k + 1) * _TK, :]
        zem = lax.dot_general(zm2, cbk, (((1,), (1,)), ((), ())),
                              preferred_element_type=jnp.float32)
        tm, ti = _scan_tile(zem, z2, e2_ref, k)
        tmins.append(tm)
        tidxs.append(ti)

    def pairmerge(va, ia, vb, ib):
        upd = vb < va
        return jnp.where(upd, vb, va), jnp.where(upd, ib, ia)

    c0v, c0i = pairmerge(tmins[0], tidxs[0], tmins[1], tidxs[1])
    c1v, c1i = pairmerge(tmins[2], tidxs[2], tmins[3], tidxs[3])
    g0v = c0v.astype(jnp.bfloat16).astype(jnp.float32)
    upd = c1v < g0v
    idx_ref[...] = jnp.where(upd, c1i, c0i)

    exact = jnp.minimum(jnp.minimum(tmins[0], tmins[1]),
                        jnp.minimum(tmins[2], tmins[3]))
    acc_ref[0] = acc_ref[0] + jnp.sum(exact)

    @pl.when(m == nm - 1)
    def _():
        vq_ref[0] = _BETA * acc_ref[0] / (_M * _D)


def _vq_search(z_e, codebook, z2, e2, ema_cluster_size):
    return pl.pallas_call(
        _vq_tc_body,
        grid=(_M // _TM,),
        in_specs=[
            pl.BlockSpec((_TM, _D), lambda m: (m, 0)),
            pl.BlockSpec((_K, _D), lambda m: (0, 0)),
            pl.BlockSpec((_TM, 1), lambda m: (m, 0)),
            pl.BlockSpec((1, _K), lambda m: (0, 0)),
            pl.BlockSpec((_K,), lambda m: (0,)),
        ],
        out_specs=[
            pl.BlockSpec((_TM, 1), lambda m: (m, 0)),
            pl.BlockSpec(memory_space=pltpu.SMEM),
            pl.BlockSpec(memory_space=pltpu.SMEM),
        ],
        out_shape=[
            jax.ShapeDtypeStruct((_M, 1), jnp.int32),
            jax.ShapeDtypeStruct((1,), jnp.float32),
            jax.ShapeDtypeStruct((1,), jnp.float32),
        ],
        scratch_shapes=[
            pltpu.SMEM((1,), jnp.float32),
        ],
        compiler_params=pltpu.CompilerParams(
            dimension_semantics=("arbitrary",)),
    )((-2.0 * z_e).astype(jnp.bfloat16), codebook.astype(jnp.bfloat16),
      z2, e2, ema_cluster_size)


_DP = 128


def _sc_gather(table_pad, indices):
    info = plsc.get_sparse_core_info()
    nc, ns = info.num_cores, info.num_subcores
    nw = nc * ns
    b_per_w = _M // nw
    mesh = plsc.VectorSubcoreMesh(core_axis_name="c", subcore_axis_name="s")

    @functools.partial(
        pl.kernel, mesh=mesh,
        out_type=jax.ShapeDtypeStruct((_M, _DP), jnp.float32),
        scratch_types=[
            pltpu.VMEM((b_per_w,), jnp.int32),
            pltpu.VMEM((b_per_w, _DP), jnp.float32),
            pltpu.SemaphoreType.DMA,
        ],
    )
    def gather_k(table_hbm, idx_hbm, out_hbm, idx_v, rows_v, sem):
        wid = lax.axis_index("s") * nc + lax.axis_index("c")
        base = wid * b_per_w
        pltpu.sync_copy(idx_hbm.at[pl.ds(base, b_per_w)], idx_v)
        pltpu.async_copy(table_hbm.at[idx_v], rows_v, sem).wait()
        pltpu.sync_copy(rows_v, out_hbm.at[pl.ds(base, b_per_w)])

    return gather_k(table_pad, indices)


def kernel(z_e, codebook, ema_cluster_size):
    z2 = (z_e ** 2).sum(axis=1, keepdims=True)
    e2 = (codebook ** 2).sum(axis=1)[None, :]
    idx2d, vq_v, ent_v = _vq_search(z_e, codebook, z2, e2, ema_cluster_size)
    indices = idx2d[:, 0]
    cb_pad = jnp.pad(codebook, ((0, 0), (0, _DP - _D)))
    z_q = _sc_gather(cb_pad, indices)[:, :_D]
    vq_loss = vq_v[0]
    perplexity = jnp.exp(ent_v[0])
    return (z_q, indices, vq_loss, perplexity, z_q)
